# Initial kernel scaffold; baseline (speedup 1.0000x reference)
import jax
import jax.numpy as jnp
from jax.experimental import pallas as pl


def kernel(x, W_width, batch_max_seq_len):
    raise NotImplementedError("write your pallas kernel here")


# speedup vs baseline: 1.0519x; 1.0519x over previous
import functools

import jax
import jax.numpy as jnp
from jax import lax
from jax.experimental import pallas as pl
from jax.experimental.pallas import tpu as pltpu
from jax.experimental.pallas import tpu_sc as plsc

WMAX = 8


@functools.lru_cache(maxsize=None)
def _build(S, D, wdim, nw):
    nspans = WMAX * S - (WMAX * (WMAX - 1)) // 2
    outw = 2 * D + wdim

    chunks_pw = nw // WMAX
    chunk = -(-S // chunks_pw)
    TS = 32
    nsub = chunk // TS

    NC = plsc.get_sparse_core_info().num_cores
    mesh = plsc.VectorSubcoreMesh(core_axis_name="c", subcore_axis_name="s")

    @functools.partial(
        pl.kernel,
        mesh=mesh,
        out_type=jax.ShapeDtypeStruct((nspans, outw), jnp.float32),
        compiler_params=pltpu.CompilerParams(
            use_tc_tiling_on_sc=False, needs_layout_passes=False
        ),
        scratch_types=[
            pltpu.VMEM((14, wdim), jnp.float32),
            pltpu.VMEM((TS, outw), jnp.float32),
            pltpu.SemaphoreType.DMA,
        ],
    )
    def spans(x_hbm, w_hbm, out_hbm, wtab_v, buf_v, sem):
        c = lax.axis_index("c")
        s = lax.axis_index("s")
        wid = s * NC + c
        w0 = wid // chunks_pw
        k = wid % chunks_pw
        cnt = (S + 1) - (w0 + 1)
        local = jnp.minimum(k * chunk, cnt - chunk)
        off = w0 * (S + 1) - (w0 * (w0 + 1)) // 2
        o = off + local
        b = w0 + 1 - (w0 >= 5).astype(jnp.int32)

        pltpu.sync_copy(w_hbm, wtab_v)
        lanes = lax.iota(jnp.int32, 16)
        brow = jnp.full((16,), b, dtype=jnp.int32)
        vw_lo = plsc.load_gather(wtab_v, [brow, lanes])
        vw_hi = plsc.load_gather(wtab_v, [brow, lanes + (wdim - 16)])
        for r in range(TS):
            rvec = jnp.full((16,), r, dtype=jnp.int32)
            plsc.store_scatter(buf_v, [rvec, lanes + 2 * D], vw_lo)
            plsc.store_scatter(buf_v, [rvec, lanes + (2 * D + wdim - 16)], vw_hi)

        def body(t, _):
            row = o + t * TS
            pltpu.sync_copy(
                x_hbm.at[pl.ds(local + t * TS, TS)],
                buf_v.at[:, pl.ds(0, D)],
            )
            pltpu.sync_copy(
                x_hbm.at[pl.ds(local + w0 + t * TS, TS)],
                buf_v.at[:, pl.ds(D, D)],
            )
            pltpu.sync_copy(buf_v, out_hbm.at[pl.ds(row, TS)])
            return 0

        lax.fori_loop(0, nsub, body, 0)

    return spans


def kernel(x, W_width, batch_max_seq_len):
    B, S, D = x.shape
    wdim = W_width.shape[1]
    nspans = WMAX * S - (WMAX * (WMAX - 1)) // 2
    info = plsc.get_sparse_core_info()
    out = _build(S, D, wdim, info.num_cores * info.num_subcores)(
        x.reshape(S, D), W_width
    )
    return out.reshape(B, nspans, 2 * D + wdim)


# speedup vs baseline: 1.0719x; 1.0190x over previous
import functools

import jax
import jax.numpy as jnp
from jax import lax
from jax.experimental import pallas as pl
from jax.experimental.pallas import tpu as pltpu
from jax.experimental.pallas import tpu_sc as plsc

WMAX = 8


@functools.lru_cache(maxsize=None)
def _build(S, D, wdim, nw):
    nspans = WMAX * S - (WMAX * (WMAX - 1)) // 2
    outw = 2 * D + wdim

    chunks_pw = nw // WMAX
    chunk = -(-S // chunks_pw)
    TS = 32
    nsub = chunk // TS

    NC = plsc.get_sparse_core_info().num_cores
    mesh = plsc.VectorSubcoreMesh(core_axis_name="c", subcore_axis_name="s")

    @functools.partial(
        pl.kernel,
        mesh=mesh,
        out_type=jax.ShapeDtypeStruct((nspans, outw), jnp.float32),
        compiler_params=pltpu.CompilerParams(
            use_tc_tiling_on_sc=False, needs_layout_passes=False
        ),
        scratch_types=[
            pltpu.VMEM((14, wdim), jnp.float32),
            pltpu.VMEM((TS, outw), jnp.float32),
            pltpu.VMEM((TS, outw), jnp.float32),
            pltpu.SemaphoreType.DMA,
            pltpu.SemaphoreType.DMA,
            pltpu.SemaphoreType.DMA,
            pltpu.SemaphoreType.DMA,
        ],
    )
    def spans(x_hbm, w_hbm, out_hbm, wtab_v, buf0_v, buf1_v, si0, si1, so0, so1):
        c = lax.axis_index("c")
        s = lax.axis_index("s")
        wid = s * NC + c
        w0 = wid // chunks_pw
        k = wid % chunks_pw
        cnt = (S + 1) - (w0 + 1)
        local = jnp.minimum(k * chunk, cnt - chunk)
        off = w0 * (S + 1) - (w0 * (w0 + 1)) // 2
        o = off + local
        b = w0 + 1 - (w0 >= 5).astype(jnp.int32)

        pltpu.sync_copy(w_hbm, wtab_v)
        lanes = lax.iota(jnp.int32, 16)
        brow = jnp.full((16,), b, dtype=jnp.int32)
        vw_lo = plsc.load_gather(wtab_v, [brow, lanes])
        vw_hi = plsc.load_gather(wtab_v, [brow, lanes + (wdim - 16)])
        bufs = (buf0_v, buf1_v)
        for buf in bufs:
            for r in range(TS):
                rvec = jnp.full((16,), r, dtype=jnp.int32)
                plsc.store_scatter(buf, [rvec, lanes + 2 * D], vw_lo)
                plsc.store_scatter(buf, [rvec, lanes + (2 * D + wdim - 16)], vw_hi)

        sin = (si0, si1)
        sout = (so0, so1)

        def start_in(t, sl):
            hl = pltpu.async_copy(
                x_hbm.at[pl.ds(local + t * TS, TS)],
                bufs[sl].at[:, pl.ds(0, D)],
                sin[sl],
            )
            hr = pltpu.async_copy(
                x_hbm.at[pl.ds(local + w0 + t * TS, TS)],
                bufs[sl].at[:, pl.ds(D, D)],
                sin[sl],
            )
            return hl, hr

        pend_in = [None, None]
        pend_out = [None, None]
        pend_in[0] = start_in(0, 0)
        for t in range(nsub):
            sl = t & 1
            hl, hr = pend_in[sl]
            hl.wait()
            hr.wait()
            pend_out[sl] = pltpu.async_copy(
                bufs[sl], out_hbm.at[pl.ds(o + t * TS, TS)], sout[sl]
            )
            if t + 1 < nsub:
                osl = 1 - sl
                if pend_out[osl] is not None:
                    pend_out[osl].wait()
                pend_in[osl] = start_in(t + 1, osl)
        pend_out[0].wait()
        pend_out[1].wait()

    return spans


def kernel(x, W_width, batch_max_seq_len):
    B, S, D = x.shape
    wdim = W_width.shape[1]
    nspans = WMAX * S - (WMAX * (WMAX - 1)) // 2
    info = plsc.get_sparse_core_info()
    out = _build(S, D, wdim, info.num_cores * info.num_subcores)(
        x.reshape(S, D), W_width
    )
    return out.reshape(B, nspans, 2 * D + wdim)


# speedup vs baseline: 1.4254x; 1.3298x over previous
import functools

import jax
import jax.numpy as jnp
from jax import lax
from jax.experimental import pallas as pl
from jax.experimental.pallas import tpu as pltpu
from jax.experimental.pallas import tpu_sc as plsc

WMAX = 8


@functools.lru_cache(maxsize=None)
def _build(S, D, wdim, nw):
    nspans = WMAX * S - (WMAX * (WMAX - 1)) // 2
    outw = 2 * D + wdim
    offs = [w0 * (S + 1) - (w0 * (w0 + 1)) // 2 for w0 in range(WMAX)]

    TS = 32
    nfull = nspans // TS
    tail = nspans - nfull * TS
    iters = -(-nfull // nw)

    NC = plsc.get_sparse_core_info().num_cores
    mesh = plsc.VectorSubcoreMesh(core_axis_name="c", subcore_axis_name="s")

    @functools.partial(
        pl.kernel,
        mesh=mesh,
        out_type=jax.ShapeDtypeStruct((nspans, outw), jnp.float32),
        compiler_params=pltpu.CompilerParams(
            use_tc_tiling_on_sc=True, needs_layout_passes=False
        ),
        scratch_types=[
            pltpu.VMEM((14, wdim), jnp.float32),
            pltpu.VMEM((TS, outw), jnp.float32),
            pltpu.VMEM((TS, outw), jnp.float32),
            pltpu.VMEM((tail, outw), jnp.float32),
            pltpu.VMEM((TS,), jnp.int32),
            pltpu.VMEM((TS,), jnp.int32),
            pltpu.VMEM((TS,), jnp.int32),
            pltpu.VMEM((TS,), jnp.int32),
            pltpu.VMEM((tail,), jnp.int32),
            pltpu.VMEM((tail,), jnp.int32),
            pltpu.SemaphoreType.DMA,
            pltpu.SemaphoreType.DMA,
            pltpu.SemaphoreType.DMA,
            pltpu.SemaphoreType.DMA,
            pltpu.SemaphoreType.DMA,
        ],
    )
    def spans(
        x_hbm, w_hbm, out_hbm, wtab_v,
        buf0_v, buf1_v, buft_v,
        il0_v, ir0_v, il1_v, ir1_v, ilt_v, irt_v,
        si0, si1, so0, so1, st,
    ):
        c = lax.axis_index("c")
        s = lax.axis_index("s")
        wid = s * NC + c
        lanes = lax.iota(jnp.int32, 16)

        pltpu.sync_copy(w_hbm, wtab_v)

        bufs = (buf0_v, buf1_v)
        idxl = (il0_v, il1_v)
        idxr = (ir0_v, ir1_v)
        sin = (si0, si1)
        sout = (so0, so1)

        def span_math(rvec):
            w0 = jnp.zeros((16,), jnp.int32)
            for j in range(1, WMAX):
                w0 = w0 + (rvec >= offs[j]).astype(jnp.int32)
            off = w0 * (S + 1) - (w0 * (w0 + 1)) // 2
            left = rvec - off
            right = left + w0
            b = w0 + 1 - (w0 >= 5).astype(jnp.int32)
            return left, right, b

        def fill_subtile(t_i, sl):
            rowbase = TS * t_i
            handles = []
            for h in range(TS // 16):
                rvec = rowbase + h * 16 + lanes
                left, right, b = span_math(rvec)
                idxl[sl][pl.ds(h * 16, 16)] = left
                idxr[sl][pl.ds(h * 16, 16)] = right
                rloc = h * 16 + lanes
                for col in range(wdim):
                    cvec = jnp.full((16,), col, dtype=jnp.int32)
                    vals = plsc.load_gather(wtab_v, [b, cvec])
                    plsc.store_scatter(
                        bufs[sl], [rloc, jnp.full((16,), 2 * D + col, jnp.int32)], vals
                    )
            handles.append(
                pltpu.async_copy(
                    x_hbm.at[idxl[sl]], bufs[sl].at[:, pl.ds(0, D)], sin[sl]
                )
            )
            handles.append(
                pltpu.async_copy(
                    x_hbm.at[idxr[sl]], bufs[sl].at[:, pl.ds(D, D)], sin[sl]
                )
            )
            return handles

        def sub_idx(i):
            return jnp.minimum(wid + nw * i, nfull - 1)

        pend_in = [None, None]
        pend_out = [None, None]
        pend_in[0] = fill_subtile(sub_idx(0), 0)
        for i in range(iters):
            sl = i & 1
            for h in pend_in[sl]:
                h.wait()
            pend_out[sl] = pltpu.async_copy(
                bufs[sl], out_hbm.at[pl.ds(TS * sub_idx(i), TS)], sout[sl]
            )
            if i + 1 < iters:
                osl = 1 - sl
                if pend_out[osl] is not None:
                    pend_out[osl].wait()
                pend_in[osl] = fill_subtile(sub_idx(i + 1), osl)
        pend_out[0].wait()
        pend_out[1].wait()

        @pl.when(wid == 0)
        def _():
            rowbase = nfull * TS
            rvec = rowbase + lanes
            left, right, b = span_math(rvec)
            tmask = lanes < tail
            plsc.store_scatter(ilt_v, [lanes], left, mask=tmask)
            plsc.store_scatter(irt_v, [lanes], right, mask=tmask)
            for col in range(wdim):
                cvec = jnp.full((16,), col, dtype=jnp.int32)
                vals = plsc.load_gather(wtab_v, [b, cvec])
                plsc.store_scatter(
                    buft_v,
                    [lanes, jnp.full((16,), 2 * D + col, jnp.int32)],
                    vals,
                    mask=tmask,
                )
            hl = pltpu.async_copy(
                x_hbm.at[ilt_v], buft_v.at[:, pl.ds(0, D)], st
            )
            hr = pltpu.async_copy(
                x_hbm.at[irt_v], buft_v.at[:, pl.ds(D, D)], st
            )
            hl.wait()
            hr.wait()
            pltpu.sync_copy(buft_v, out_hbm.at[pl.ds(rowbase, tail)])

    return spans


def kernel(x, W_width, batch_max_seq_len):
    B, S, D = x.shape
    wdim = W_width.shape[1]
    nspans = WMAX * S - (WMAX * (WMAX - 1)) // 2
    info = plsc.get_sparse_core_info()
    out = _build(S, D, wdim, info.num_cores * info.num_subcores)(
        x.reshape(S, D), W_width
    )
    return out.reshape(B, nspans, 2 * D + wdim)


# speedup vs baseline: 2.4054x; 1.6875x over previous
import functools

import jax
import jax.numpy as jnp
from jax import lax
from jax.experimental import pallas as pl
from jax.experimental.pallas import tpu as pltpu
from jax.experimental.pallas import tpu_sc as plsc

WMAX = 8


@functools.lru_cache(maxsize=None)
def _build(S, D, wdim, nw):
    nspans = WMAX * S - (WMAX * (WMAX - 1)) // 2
    outw = 2 * D + wdim
    offs = [w0 * (S + 1) - (w0 * (w0 + 1)) // 2 for w0 in range(WMAX)]

    TS = 32
    nfull = nspans // TS
    tail = nspans - nfull * TS
    iters = -(-nfull // nw)

    NC = plsc.get_sparse_core_info().num_cores
    mesh = plsc.VectorSubcoreMesh(core_axis_name="c", subcore_axis_name="s")

    @functools.partial(
        pl.kernel,
        mesh=mesh,
        out_type=jax.ShapeDtypeStruct((1, nspans, outw), jnp.float32),
        compiler_params=pltpu.CompilerParams(
            use_tc_tiling_on_sc=True, needs_layout_passes=False
        ),
        scratch_types=[
            pltpu.VMEM((14, wdim), jnp.float32),
            pltpu.VMEM((TS, outw), jnp.float32),
            pltpu.VMEM((TS, outw), jnp.float32),
            pltpu.VMEM((tail, outw), jnp.float32),
            pltpu.VMEM((TS,), jnp.int32),
            pltpu.VMEM((TS,), jnp.int32),
            pltpu.VMEM((TS,), jnp.int32),
            pltpu.VMEM((TS,), jnp.int32),
            pltpu.VMEM((tail,), jnp.int32),
            pltpu.VMEM((tail,), jnp.int32),
            pltpu.SemaphoreType.DMA,
            pltpu.SemaphoreType.DMA,
            pltpu.SemaphoreType.DMA,
            pltpu.SemaphoreType.DMA,
            pltpu.SemaphoreType.DMA,
        ],
    )
    def spans(
        x_hbm, w_hbm, out_hbm, wtab_v,
        buf0_v, buf1_v, buft_v,
        il0_v, ir0_v, il1_v, ir1_v, ilt_v, irt_v,
        si0, si1, so0, so1, st,
    ):
        c = lax.axis_index("c")
        s = lax.axis_index("s")
        wid = s * NC + c
        lanes = lax.iota(jnp.int32, 16)

        pltpu.sync_copy(w_hbm, wtab_v)

        bufs = (buf0_v, buf1_v)
        idxl = (il0_v, il1_v)
        idxr = (ir0_v, ir1_v)
        sin = (si0, si1)
        sout = (so0, so1)

        def span_math(rvec):
            w0 = jnp.zeros((16,), jnp.int32)
            for j in range(1, WMAX):
                w0 = w0 + (rvec >= offs[j]).astype(jnp.int32)
            off = w0 * (S + 1) - (w0 * (w0 + 1)) // 2
            left = rvec - off
            right = left + w0
            b = w0 + 1 - (w0 >= 5).astype(jnp.int32)
            return left, right, b

        def fill_subtile(t_i, sl):
            rowbase = TS * t_i
            handles = []
            for h in range(TS // 16):
                rvec = rowbase + h * 16 + lanes
                left, right, b = span_math(rvec)
                idxl[sl][pl.ds(h * 16, 16)] = left
                idxr[sl][pl.ds(h * 16, 16)] = right
                rloc = h * 16 + lanes
                for col in range(wdim):
                    cvec = jnp.full((16,), col, dtype=jnp.int32)
                    vals = plsc.load_gather(wtab_v, [b, cvec])
                    plsc.store_scatter(
                        bufs[sl], [rloc, jnp.full((16,), 2 * D + col, jnp.int32)], vals
                    )
            handles.append(
                pltpu.async_copy(
                    x_hbm.at[idxl[sl]], bufs[sl].at[:, pl.ds(0, D)], sin[sl]
                )
            )
            handles.append(
                pltpu.async_copy(
                    x_hbm.at[idxr[sl]], bufs[sl].at[:, pl.ds(D, D)], sin[sl]
                )
            )
            return handles

        def sub_idx(i):
            return jnp.minimum(wid + nw * i, nfull - 1)

        pend_in = [None, None]
        pend_out = [None, None]
        pend_in[0] = fill_subtile(sub_idx(0), 0)
        for i in range(iters):
            sl = i & 1
            for h in pend_in[sl]:
                h.wait()
            pend_out[sl] = pltpu.async_copy(
                bufs[sl], out_hbm.at[0, pl.ds(TS * sub_idx(i), TS)], sout[sl]
            )
            if i + 1 < iters:
                osl = 1 - sl
                if pend_out[osl] is not None:
                    pend_out[osl].wait()
                pend_in[osl] = fill_subtile(sub_idx(i + 1), osl)
        pend_out[0].wait()
        pend_out[1].wait()

        @pl.when(wid == 0)
        def _():
            rowbase = nfull * TS
            rvec = rowbase + lanes
            left, right, b = span_math(rvec)
            tmask = lanes < tail
            plsc.store_scatter(ilt_v, [lanes], left, mask=tmask)
            plsc.store_scatter(irt_v, [lanes], right, mask=tmask)
            for col in range(wdim):
                cvec = jnp.full((16,), col, dtype=jnp.int32)
                vals = plsc.load_gather(wtab_v, [b, cvec])
                plsc.store_scatter(
                    buft_v,
                    [lanes, jnp.full((16,), 2 * D + col, jnp.int32)],
                    vals,
                    mask=tmask,
                )
            hl = pltpu.async_copy(
                x_hbm.at[ilt_v], buft_v.at[:, pl.ds(0, D)], st
            )
            hr = pltpu.async_copy(
                x_hbm.at[irt_v], buft_v.at[:, pl.ds(D, D)], st
            )
            hl.wait()
            hr.wait()
            pltpu.sync_copy(buft_v, out_hbm.at[0, pl.ds(rowbase, tail)])

    return spans


def kernel(x, W_width, batch_max_seq_len):
    B, S, D = x.shape
    wdim = W_width.shape[1]
    info = plsc.get_sparse_core_info()
    return _build(S, D, wdim, info.num_cores * info.num_subcores)(
        x.reshape(S, D), W_width
    )
